# Initial kernel scaffold; baseline (speedup 1.0000x reference)
import jax
import jax.numpy as jnp
from jax.experimental import pallas as pl


def kernel(weights):
    raise NotImplementedError("write your pallas kernel here")


# speedup vs baseline: 444.2747x; 444.2747x over previous
import functools

import jax
import jax.numpy as jnp
from jax.experimental import pallas as pl
from jax.experimental.pallas import tpu as pltpu

_BM = 256


def _nm24_body(x_ref, o_ref):
    x = x_ref[...]
    a = jnp.abs(x)
    p = jax.lax.broadcasted_iota(jnp.int32, x.shape, 1) & 3

    rank = jnp.zeros(x.shape, jnp.int32)
    for d in (1, 2, 3):
        cg = (pltpu.roll(a, x.shape[1] - d, 1) > a).astype(jnp.int32)
        fwd = jnp.where(p < 4 - d, cg, 0)
        back = jnp.where(p >= d, 1 - pltpu.roll(cg, d, 1), 0)
        rank = rank + fwd + back

    o_ref[...] = jnp.where(rank < 2, x, jnp.zeros_like(x))


def _nm24(weights):
    m, n = weights.shape
    grid = (m // _BM,)
    return pl.pallas_call(
        _nm24_body,
        grid=grid,
        in_specs=[pl.BlockSpec((_BM, n), lambda i: (i, 0))],
        out_specs=pl.BlockSpec((_BM, n), lambda i: (i, 0)),
        out_shape=jax.ShapeDtypeStruct((m, n), weights.dtype),
    )(weights)


@jax.jit
def kernel(weights):
    return _nm24(weights)
